# baseline (speedup 1.0000x reference)
import functools

import jax
import jax.numpy as jnp
from jax import lax
from jax.experimental import pallas as pl
from jax.experimental.pallas import tpu as pltpu
from jax.experimental.pallas import tpu_sc as plsc

_B = 16384
_A = 4
_NC = 2
_NS = 16
_L = 16
_NW = _NC * _NS
_CHUNK = _B // _NW
_ITERS = _CHUNK // _L


def _sc_body(u_hbm, pmf_hbm, cho_hbm, u_v, pmf_v, cho_v):
    wid = lax.axis_index("s") * _NC + lax.axis_index("c")
    base = wid * _CHUNK
    pltpu.sync_copy(u_hbm.at[pl.ds(base, _CHUNK)], u_v)

    quarter = jnp.full((_L,), 0.25, jnp.float32)
    half = jnp.full((_L,), 0.5, jnp.float32)
    three_q = jnp.full((_L,), 0.75, jnp.float32)
    one = jnp.full((_L,), 1.0, jnp.float32)
    zeros = jnp.zeros((_L,), jnp.int32)
    ones = jnp.full((_L,), 1, jnp.int32)

    def body(i, carry):
        s = pl.multiple_of(i * _L, _L)
        uv = u_v[pl.ds(s, _L)]
        c = lax.select(uv > quarter, ones, zeros)
        c = c + lax.select(uv > half, ones, zeros)
        c = c + lax.select(uv > three_q, ones, zeros)
        c = c + lax.select(uv > one, ones, zeros)
        cho_v[pl.ds(s, _L)] = c
        p = pl.multiple_of(i * (_L * _A), _L * _A)
        pmf_v[pl.ds(p, _L)] = quarter
        pmf_v[pl.ds(p + _L, _L)] = quarter
        pmf_v[pl.ds(p + 2 * _L, _L)] = quarter
        pmf_v[pl.ds(p + 3 * _L, _L)] = quarter
        return carry

    lax.fori_loop(0, _ITERS, body, 0)

    pltpu.sync_copy(pmf_v, pmf_hbm.at[pl.ds(base * _A, _CHUNK * _A)])
    pltpu.sync_copy(cho_v, cho_hbm.at[pl.ds(base, _CHUNK)])


_sc_call = functools.partial(
    pl.kernel,
    out_type=(
        jax.ShapeDtypeStruct((_B * _A,), jnp.float32),
        jax.ShapeDtypeStruct((_B,), jnp.int32),
    ),
    mesh=plsc.VectorSubcoreMesh(core_axis_name="c", subcore_axis_name="s"),
    scratch_types=[
        pltpu.VMEM((_CHUNK,), jnp.float32),
        pltpu.VMEM((_CHUNK * _A,), jnp.float32),
        pltpu.VMEM((_CHUNK,), jnp.int32),
    ],
)(_sc_body)


def kernel(current_states, u):
    del current_states
    pmf_flat, choices = _sc_call(u.reshape(_B))
    return pmf_flat.reshape(_B, _A), choices
